# baseline (speedup 1.0000x reference)
import functools

import jax
import jax.numpy as jnp
from jax import lax
from jax.experimental import pallas as pl
from jax.experimental.pallas import tpu as pltpu
from jax.experimental.pallas import tpu_sc as plsc

B = 1024
D = 4096
R = 5
NC = 2
NS = 16
L = 16
NW = NC * NS
RPW = B // NW
UN = 8
K = 2
BLK = K * R * D
NBLK = RPW // K
NBUF = 2

_mesh = plsc.VectorSubcoreMesh(core_axis_name="c", subcore_axis_name="s")


@functools.partial(
    pl.kernel,
    mesh=_mesh,
    out_type=(
        jax.ShapeDtypeStruct((B * R * D,), jnp.float32),
        jax.ShapeDtypeStruct((B,), jnp.float32),
    ),
    scratch_types=[
        pltpu.VMEM((RPW + L,), jnp.float32),
        pltpu.VMEM((RPW + L,), jnp.float32),
        pltpu.VMEM((NBUF, BLK), jnp.float32),
        pltpu.VMEM((RPW,), jnp.float32),
        pltpu.SemaphoreType.DMA,
        pltpu.SemaphoreType.DMA,
    ],
)
def _sc_body(b_hbm, s_hbm, mats_hbm, ones_hbm, bvs, svs, pbuf, obuf, sem0, sem1):
    wid = lax.axis_index("s") * NC + lax.axis_index("c")
    base = wid * RPW
    fi = lax.broadcasted_iota(jnp.int32, (L,), 0).astype(jnp.float32)
    zero = jnp.zeros((L,), jnp.float32)
    one = jnp.ones((L,), jnp.float32)
    sems = (sem0, sem1)

    pltpu.sync_copy(b_hbm.at[pl.ds(base, RPW)], bvs.at[pl.ds(0, RPW)])
    pltpu.sync_copy(s_hbm.at[pl.ds(base, RPW)], svs.at[pl.ds(0, RPW)])

    def zb(c, carry):
        for par in range(NBUF):
            pbuf[par, pl.ds(c * L, L)] = zero
        return carry

    lax.fori_loop(0, BLK // L, zb, 0)

    for g in range(RPW // L):
        obuf[pl.ds(g * L, L)] = one

    def fill_row(par, j, r):
        bb = jnp.full((L,), bvs[pl.ds(r, L)][0], jnp.float32)
        ss = jnp.full((L,), svs[pl.ds(r, L)][0], jnp.float32)
        v0 = bb + fi * ss
        deltas = [ss * jnp.float32(L * k) for k in range(UN)]
        stride = ss * jnp.float32(L * UN)
        jbase = j * R * D

        def chunk(c, v):
            off = jbase + c * (L * UN)
            for k in range(UN):
                pbuf[par, pl.ds(off + k * L, L)] = v + deltas[k]
            return v + stride

        lax.fori_loop(0, D // (L * UN), chunk, v0)

    def blk_body(g, carry):
        for par in range(NBUF):
            bi = g * NBUF + par
            off = (base + bi * K) * (R * D)

            @pl.when(g > 0)
            def _drain():
                pltpu.make_async_copy(
                    pbuf.at[par], mats_hbm.at[pl.ds(off, BLK)], sems[par]
                ).wait()

            for j in range(K):
                fill_row(par, j, bi * K + j)
            pltpu.async_copy(
                pbuf.at[par], mats_hbm.at[pl.ds(off, BLK)], sems[par]
            )
        return carry

    lax.fori_loop(0, NBLK // NBUF, blk_body, 0)

    for par in range(NBUF):
        last_off = (base + (NBLK - NBUF + par) * K) * (R * D)
        pltpu.make_async_copy(
            pbuf.at[par], mats_hbm.at[pl.ds(last_off, BLK)], sems[par]
        ).wait()

    pltpu.sync_copy(obuf, ones_hbm.at[pl.ds(base, RPW)])


def kernel(floats):
    b_arr = floats[:, 0]
    s_arr = (floats[:, 1] - floats[:, 0]) / jnp.float32(D - 1)
    mats_flat, ones_flat = _sc_body(b_arr, s_arr)
    return (mats_flat.reshape(B, R, D), ones_flat.reshape(B, 1))


# speedup vs baseline: 1.7066x; 1.7066x over previous
import functools

import jax
import jax.numpy as jnp
from jax import lax
from jax.experimental import pallas as pl
from jax.experimental.pallas import tpu as pltpu
from jax.experimental.pallas import tpu_sc as plsc

B = 1024
D = 4096
R = 5
NC = 2
NS = 16
L = 16
NW = NC * NS
RPW = B // NW
UN = 8
K = 1
NBLK = RPW // K
NBUF = 2

_mesh = plsc.VectorSubcoreMesh(core_axis_name="c", subcore_axis_name="s")


@functools.partial(
    pl.kernel,
    mesh=_mesh,
    out_type=(
        jax.ShapeDtypeStruct((B, R, D), jnp.float32),
        jax.ShapeDtypeStruct((B,), jnp.float32),
    ),
    scratch_types=[
        pltpu.VMEM((RPW + L,), jnp.float32),
        pltpu.VMEM((RPW + L,), jnp.float32),
        pltpu.VMEM((NBUF, K, R, D), jnp.float32),
        pltpu.VMEM((RPW,), jnp.float32),
        pltpu.SemaphoreType.DMA,
        pltpu.SemaphoreType.DMA,
    ],
)
def _sc_body(b_hbm, s_hbm, mats_hbm, ones_hbm, bvs, svs, pbuf, obuf, sem0, sem1):
    wid = lax.axis_index("s") * NC + lax.axis_index("c")
    base = wid * RPW
    fi = lax.broadcasted_iota(jnp.int32, (L,), 0).astype(jnp.float32)
    zero = jnp.zeros((L,), jnp.float32)
    one = jnp.ones((L,), jnp.float32)
    sems = (sem0, sem1)

    pltpu.sync_copy(b_hbm.at[pl.ds(base, RPW)], bvs.at[pl.ds(0, RPW)])
    pltpu.sync_copy(s_hbm.at[pl.ds(base, RPW)], svs.at[pl.ds(0, RPW)])

    def zb(c, carry):
        for par in range(NBUF):
            for kk in range(K):
                for rr in range(R):
                    pbuf[par, kk, rr, pl.ds(c * L, L)] = zero
        return carry

    lax.fori_loop(0, D // L, zb, 0)

    for g in range(RPW // L):
        obuf[pl.ds(g * L, L)] = one

    def fill_row(par, j, r):
        bb = jnp.full((L,), bvs[pl.ds(r, L)][0], jnp.float32)
        ss = jnp.full((L,), svs[pl.ds(r, L)][0], jnp.float32)
        v0 = bb + fi * ss
        deltas = [ss * jnp.float32(L * k) for k in range(UN)]
        stride = ss * jnp.float32(L * UN)

        def chunk(c, v):
            off = c * (L * UN)
            for k in range(UN):
                pbuf[par, j, 0, pl.ds(off + k * L, L)] = v + deltas[k]
            return v + stride

        lax.fori_loop(0, D // (L * UN), chunk, v0)

    def blk_body(g, carry):
        for par in range(NBUF):
            bi = g * NBUF + par
            row0 = base + bi * K

            @pl.when(g > 0)
            def _drain():
                pltpu.make_async_copy(
                    pbuf.at[par], mats_hbm.at[pl.ds(row0, K)], sems[par]
                ).wait()

            for j in range(K):
                fill_row(par, j, bi * K + j)
            pltpu.async_copy(
                pbuf.at[par], mats_hbm.at[pl.ds(row0, K)], sems[par]
            )
        return carry

    lax.fori_loop(0, NBLK // NBUF, blk_body, 0)

    for par in range(NBUF):
        last_row0 = base + (NBLK - NBUF + par) * K
        pltpu.make_async_copy(
            pbuf.at[par], mats_hbm.at[pl.ds(last_row0, K)], sems[par]
        ).wait()

    pltpu.sync_copy(obuf, ones_hbm.at[pl.ds(base, RPW)])


def kernel(floats):
    b_arr = floats[:, 0]
    s_arr = (floats[:, 1] - floats[:, 0]) / jnp.float32(D - 1)
    mats, ones_flat = _sc_body(b_arr, s_arr)
    return (mats, ones_flat.reshape(B, 1))
